# baseline (speedup 1.0000x reference)
import jax
import jax.numpy as jnp
from jax import lax
from jax.experimental import pallas as pl
from jax.experimental.pallas import tpu as pltpu
from jax.experimental.pallas import tpu_sc as plsc

_ROWS, _COLS = 1024, 100000
_SC_COLS = 99328
_NC, _NS = 2, 16
_NW = _NC * _NS
_RPW = _ROWS // _NW
_NEG = float("-inf")
_GATHER_DNUMS = lax.GatherDimensionNumbers(
    offset_dims=(), collapsed_slice_dims=(0,), start_index_map=(0,)
)


def _rot(v, iot, sh):
    idx = lax.rem(iot + sh, jnp.full((16,), 16, dtype=jnp.int32))
    return lax.gather(
        v,
        idx[:, None],
        dimension_numbers=_GATHER_DNUMS,
        slice_sizes=(1,),
        mode=lax.GatherScatterMode.PROMISE_IN_BOUNDS,
    )


def _allmax(v, iot):
    for sh in (8, 4, 2, 1):
        v = jnp.maximum(v, _rot(v, iot, sh))
    return v


_W = 1024
_NCHUNK = _SC_COLS // _W
_NGROUP = _RPW // 8


def _sc_body(x_hbm, y_hbm, outm_hbm, outc_hbm, ybuf, b0, b1, b2, b3, mbuf, cbuf, sems):
    bufs = (b0, b1, b2, b3)
    c = lax.axis_index("c")
    s = lax.axis_index("s")
    wid = s * _NC + c
    base = pl.multiple_of(wid * _RPW, _RPW)
    pltpu.sync_copy(y_hbm.at[pl.ds(base, _RPW)], ybuf)

    iot = lax.iota(jnp.int32, 16)
    neg = jnp.float32(_NEG)
    negv = jnp.full((16,), _NEG, dtype=jnp.float32)

    def start(j, row8, cb):
        pltpu.make_async_copy(
            x_hbm.at[pl.ds(row8, 8), pl.ds(pl.multiple_of(cb * _W, _W), _W)],
            bufs[j],
            sems.at[j],
        ).start()

    for g in range(_NGROUP):
        row8 = pl.multiple_of(base + 8 * g, 8)
        ybase = (8 * g) // 16 * 16
        yvec = ybuf[pl.ds(ybase, 16)]
        lane0 = (8 * g) % 16
        y_sc = [_rot(yvec, iot, lane0 + r)[0] for r in range(8)]
        cb_y = [yy // _W for yy in y_sc]
        off_y = [yy - cc * _W for yy, cc in zip(y_sc, cb_y)]

        for j in range(4):
            start(j, row8, jnp.int32(j))

        accs = [negv] * 8
        cvecs = [negv] * 8

        def macro_body(k, carry, row8=row8, cb_y=cb_y, off_y=off_y):
            lists = list(carry)
            for j in range(4):
                cb = k * 4 + j
                pltpu.make_async_copy(
                    x_hbm.at[
                        pl.ds(row8, 8), pl.ds(pl.multiple_of(cb * _W, _W), _W)
                    ],
                    bufs[j],
                    sems.at[j],
                ).wait()
                for r in range(8):
                    is_mine = cb_y[r] == cb
                    vbase = pl.multiple_of((off_y[r] // 16) * 16, 16)
                    lane2 = off_y[r] - vbase
                    v16 = bufs[j][r, pl.ds(vbase, 16)]
                    lane2m = jnp.where(is_mine, lane2, jnp.int32(-100))
                    mm = iot == lane2m
                    lists[8 + r] = jnp.where(mm, v16, lists[8 + r])
                    bufs[j][r, pl.ds(vbase, 16)] = jnp.where(mm, neg, v16)
                def chunk_max(i, aa, j=j):
                    aa = list(aa)
                    for r in range(8):
                        for u in range(2):
                            aa[r] = jnp.maximum(
                                aa[r], bufs[j][r, pl.ds((i * 2 + u) * 16, 16)]
                            )
                    return tuple(aa)

                upd = lax.fori_loop(0, _W // 32, chunk_max, tuple(lists[:8]))
                for r in range(8):
                    lists[r] = upd[r]

                @pl.when(cb + 4 < _NCHUNK)
                def _(j=j, cb=cb, row8=row8):
                    start(j, row8, cb + 4)

            return tuple(lists)

        out = lax.fori_loop(0, _NCHUNK // 4, macro_body, tuple(accs + cvecs))
        out = macro_tail(out, x_hbm, bufs, sems, iot, neg, row8, cb_y, off_y)
        accs, cvecs = list(out[:8]), list(out[8:])

        for r in range(8):
            bestv = _allmax(accs[r], iot)
            correctv = _allmax(cvecs[r], iot)
            kv = jnp.full((16,), 8 * g + r, dtype=jnp.int32)
            row_in_half = (8 * g + r) % 16
            half = (8 * g + r) // 16
            hit = iot == jnp.full((16,), row_in_half, dtype=jnp.int32)
            if half == 0:
                mbuf[pl.ds(0, 16)] = jnp.where(hit, bestv, mbuf[pl.ds(0, 16)])
                cbuf[pl.ds(0, 16)] = jnp.where(hit, correctv, cbuf[pl.ds(0, 16)])
            else:
                mbuf[pl.ds(16, 16)] = jnp.where(hit, bestv, mbuf[pl.ds(16, 16)])
                cbuf[pl.ds(16, 16)] = jnp.where(hit, correctv, cbuf[pl.ds(16, 16)])

    pltpu.sync_copy(mbuf, outm_hbm.at[pl.ds(base, _RPW)])
    pltpu.sync_copy(cbuf, outc_hbm.at[pl.ds(base, _RPW)])


def macro_tail(carry, x_hbm, bufs, sems, iot, neg, row8, cb_y, off_y):
    lists = list(carry)
    cb = _NCHUNK - 1
    pltpu.make_async_copy(
        x_hbm.at[pl.ds(row8, 8), pl.ds(pl.multiple_of(cb * _W, _W), _W)],
        bufs[0],
        sems.at[0],
    ).wait()
    for r in range(8):
        is_mine = cb_y[r] == cb
        vbase = pl.multiple_of((off_y[r] // 16) * 16, 16)
        lane2 = off_y[r] - vbase
        v16 = bufs[0][r, pl.ds(vbase, 16)]
        lane2m = jnp.where(is_mine, lane2, jnp.int32(-100))
        mm = iot == lane2m
        lists[8 + r] = jnp.where(mm, v16, lists[8 + r])
        bufs[0][r, pl.ds(vbase, 16)] = jnp.where(mm, neg, v16)
    def chunk_max(i, aa):
        aa = list(aa)
        for r in range(8):
            for u in range(2):
                aa[r] = jnp.maximum(aa[r], bufs[0][r, pl.ds((i * 2 + u) * 16, 16)])
        return tuple(aa)

    upd = lax.fori_loop(0, _W // 32, chunk_max, tuple(lists[:8]))
    for r in range(8):
        lists[r] = upd[r]
    return tuple(lists)


_MESH = plsc.VectorSubcoreMesh(
    core_axis_name="c", subcore_axis_name="s", num_cores=_NC, num_subcores=_NS
)

_TC_BLOCK_ROWS = 64
_TC_GRID = _ROWS // _TC_BLOCK_ROWS
_TC_COL_BLOCK = 1024
_TC_COL_INDEX = _SC_COLS // _TC_COL_BLOCK


def _tc_tail_kernel(y_ref, x_ref, m_ref, c_ref):
    x = x_ref[...]
    yb = y_ref[0, 0, :]
    col = _SC_COLS + jax.lax.broadcasted_iota(jnp.int32, x.shape, 1)
    is_y = col == yb[:, None]
    valid = col < _COLS
    neg = jnp.float32(-jnp.inf)
    m_ref[0, 0, :] = jnp.max(jnp.where(valid & (~is_y), x, neg), axis=-1)
    c_ref[0, 0, :] = jnp.max(jnp.where(valid & is_y, x, neg), axis=-1)


@jax.jit
def kernel(logits, y):
    y32 = y.astype(jnp.int32)
    m_sc, c_sc = pl.kernel(
        _sc_body,
        out_type=(
            jax.ShapeDtypeStruct((_ROWS,), jnp.float32),
            jax.ShapeDtypeStruct((_ROWS,), jnp.float32),
        ),
        mesh=_MESH,
        scratch_types=[
            pltpu.VMEM((_RPW,), jnp.int32),
            pltpu.VMEM((8, _W), jnp.float32),
            pltpu.VMEM((8, _W), jnp.float32),
            pltpu.VMEM((8, _W), jnp.float32),
            pltpu.VMEM((8, _W), jnp.float32),
            pltpu.VMEM((_RPW,), jnp.float32),
            pltpu.VMEM((_RPW,), jnp.float32),
            pltpu.SemaphoreType.DMA((4,)),
        ],
    )(logits, y32)

    y3 = y32.reshape(_TC_GRID, 1, _TC_BLOCK_ROWS)
    m_tc, c_tc = pl.pallas_call(
        _tc_tail_kernel,
        grid=(_TC_GRID,),
        in_specs=[
            pl.BlockSpec((1, 1, _TC_BLOCK_ROWS), lambda i: (i, 0, 0)),
            pl.BlockSpec((_TC_BLOCK_ROWS, _TC_COL_BLOCK), lambda i: (i, _TC_COL_INDEX)),
        ],
        out_specs=[
            pl.BlockSpec((1, 1, _TC_BLOCK_ROWS), lambda i: (i, 0, 0)),
            pl.BlockSpec((1, 1, _TC_BLOCK_ROWS), lambda i: (i, 0, 0)),
        ],
        out_shape=[
            jax.ShapeDtypeStruct((_TC_GRID, 1, _TC_BLOCK_ROWS), jnp.float32),
            jax.ShapeDtypeStruct((_TC_GRID, 1, _TC_BLOCK_ROWS), jnp.float32),
        ],
    )(y3, logits)
    m_tc = m_tc.reshape(_ROWS)
    c_tc = c_tc.reshape(_ROWS)

    correct = jnp.maximum(c_sc, c_tc)
    best_other = jnp.maximum(m_sc, m_tc)
    return jnp.maximum(correct - best_other, jnp.float32(0.0))


# speedup vs baseline: 1.0406x; 1.0406x over previous
import jax
import jax.numpy as jnp
from jax import lax
from jax.experimental import pallas as pl
from jax.experimental.pallas import tpu as pltpu
from jax.experimental.pallas import tpu_sc as plsc

_ROWS, _COLS = 1024, 100000
_SC_COLS = 99328
_NC, _NS = 2, 16
_NW = _NC * _NS
_SC_ROWS = 512
_RPW = _SC_ROWS // _NW
_NEG = float("-inf")
_GATHER_DNUMS = lax.GatherDimensionNumbers(
    offset_dims=(), collapsed_slice_dims=(0,), start_index_map=(0,)
)


def _rot(v, iot, sh):
    idx = lax.rem(iot + sh, jnp.full((16,), 16, dtype=jnp.int32))
    return lax.gather(
        v,
        idx[:, None],
        dimension_numbers=_GATHER_DNUMS,
        slice_sizes=(1,),
        mode=lax.GatherScatterMode.PROMISE_IN_BOUNDS,
    )


def _allmax(v, iot):
    for sh in (8, 4, 2, 1):
        v = jnp.maximum(v, _rot(v, iot, sh))
    return v


_W = 1024
_NCHUNK = _SC_COLS // _W
_NGROUP = _RPW // 8


def _sc_body(x_hbm, y_hbm, outm_hbm, outc_hbm, ybuf, b0, b1, b2, b3, mbuf, cbuf, sems):
    bufs = (b0, b1, b2, b3)
    c = lax.axis_index("c")
    s = lax.axis_index("s")
    wid = s * _NC + c
    base = pl.multiple_of(wid * _RPW, _RPW)
    pltpu.sync_copy(y_hbm.at[pl.ds(base, _RPW)], ybuf)

    iot = lax.iota(jnp.int32, 16)
    neg = jnp.float32(_NEG)
    negv = jnp.full((16,), _NEG, dtype=jnp.float32)

    def start(j, row8, cb):
        pltpu.make_async_copy(
            x_hbm.at[pl.ds(row8, 8), pl.ds(pl.multiple_of(cb * _W, _W), _W)],
            bufs[j],
            sems.at[j],
        ).start()

    for g in range(_NGROUP):
        row8 = pl.multiple_of(base + 8 * g, 8)
        ybase = (8 * g) // 16 * 16
        yvec = ybuf[pl.ds(ybase, 16)]
        lane0 = (8 * g) % 16
        y_sc = [_rot(yvec, iot, lane0 + r)[0] for r in range(8)]
        cb_y = [yy // _W for yy in y_sc]
        off_y = [yy - cc * _W for yy, cc in zip(y_sc, cb_y)]

        for j in range(4):
            start(j, row8, jnp.int32(j))

        accs = [negv] * 8
        cvecs = [negv] * 8

        def macro_body(k, carry, row8=row8, cb_y=cb_y, off_y=off_y):
            lists = list(carry)
            for j in range(4):
                cb = k * 4 + j
                pltpu.make_async_copy(
                    x_hbm.at[
                        pl.ds(row8, 8), pl.ds(pl.multiple_of(cb * _W, _W), _W)
                    ],
                    bufs[j],
                    sems.at[j],
                ).wait()
                for r in range(8):
                    is_mine = cb_y[r] == cb
                    vbase = pl.multiple_of((off_y[r] // 16) * 16, 16)
                    lane2 = off_y[r] - vbase
                    v16 = bufs[j][r, pl.ds(vbase, 16)]
                    lane2m = jnp.where(is_mine, lane2, jnp.int32(-100))
                    mm = iot == lane2m
                    lists[8 + r] = jnp.where(mm, v16, lists[8 + r])
                    bufs[j][r, pl.ds(vbase, 16)] = jnp.where(mm, neg, v16)
                def chunk_max(i, aa, j=j):
                    aa = list(aa)
                    for r in range(8):
                        for u in range(2):
                            aa[r] = jnp.maximum(
                                aa[r], bufs[j][r, pl.ds((i * 2 + u) * 16, 16)]
                            )
                    return tuple(aa)

                upd = lax.fori_loop(0, _W // 32, chunk_max, tuple(lists[:8]))
                for r in range(8):
                    lists[r] = upd[r]

                @pl.when(cb + 4 < _NCHUNK)
                def _(j=j, cb=cb, row8=row8):
                    start(j, row8, cb + 4)

            return tuple(lists)

        out = lax.fori_loop(0, _NCHUNK // 4, macro_body, tuple(accs + cvecs))
        out = macro_tail(out, x_hbm, bufs, sems, iot, neg, row8, cb_y, off_y)
        accs, cvecs = list(out[:8]), list(out[8:])

        for r in range(8):
            bestv = _allmax(accs[r], iot)
            correctv = _allmax(cvecs[r], iot)
            kv = jnp.full((16,), 8 * g + r, dtype=jnp.int32)
            row_in_half = (8 * g + r) % 16
            half = (8 * g + r) // 16
            hit = iot == jnp.full((16,), row_in_half, dtype=jnp.int32)
            if half == 0:
                mbuf[pl.ds(0, 16)] = jnp.where(hit, bestv, mbuf[pl.ds(0, 16)])
                cbuf[pl.ds(0, 16)] = jnp.where(hit, correctv, cbuf[pl.ds(0, 16)])
            else:
                mbuf[pl.ds(16, 16)] = jnp.where(hit, bestv, mbuf[pl.ds(16, 16)])
                cbuf[pl.ds(16, 16)] = jnp.where(hit, correctv, cbuf[pl.ds(16, 16)])

    pltpu.sync_copy(mbuf, outm_hbm.at[pl.ds(base, _RPW)])
    pltpu.sync_copy(cbuf, outc_hbm.at[pl.ds(base, _RPW)])


def macro_tail(carry, x_hbm, bufs, sems, iot, neg, row8, cb_y, off_y):
    lists = list(carry)
    cb = _NCHUNK - 1
    pltpu.make_async_copy(
        x_hbm.at[pl.ds(row8, 8), pl.ds(pl.multiple_of(cb * _W, _W), _W)],
        bufs[0],
        sems.at[0],
    ).wait()
    for r in range(8):
        is_mine = cb_y[r] == cb
        vbase = pl.multiple_of((off_y[r] // 16) * 16, 16)
        lane2 = off_y[r] - vbase
        v16 = bufs[0][r, pl.ds(vbase, 16)]
        lane2m = jnp.where(is_mine, lane2, jnp.int32(-100))
        mm = iot == lane2m
        lists[8 + r] = jnp.where(mm, v16, lists[8 + r])
        bufs[0][r, pl.ds(vbase, 16)] = jnp.where(mm, neg, v16)
    def chunk_max(i, aa):
        aa = list(aa)
        for r in range(8):
            for u in range(2):
                aa[r] = jnp.maximum(aa[r], bufs[0][r, pl.ds((i * 2 + u) * 16, 16)])
        return tuple(aa)

    upd = lax.fori_loop(0, _W // 32, chunk_max, tuple(lists[:8]))
    for r in range(8):
        lists[r] = upd[r]
    return tuple(lists)


_MESH = plsc.VectorSubcoreMesh(
    core_axis_name="c", subcore_axis_name="s", num_cores=_NC, num_subcores=_NS
)

_TC_BLOCK_ROWS = 64
_TC_GRID = _SC_ROWS // _TC_BLOCK_ROWS
_TC_HI_GRID = (_ROWS - _SC_ROWS) // _TC_BLOCK_ROWS
_TC_COL_BLOCK = 1024
_TC_COL_INDEX = _SC_COLS // _TC_COL_BLOCK


def _tc_full_kernel(y_ref, x_ref, out_ref):
    x = x_ref[...]
    yb = y_ref[0, 0, :]
    col = jax.lax.broadcasted_iota(jnp.int32, x.shape, 1)
    is_y = col == yb[:, None]
    neg = jnp.float32(-jnp.inf)
    best_other = jnp.max(jnp.where(is_y, neg, x), axis=-1)
    correct = jnp.max(jnp.where(is_y, x, neg), axis=-1)
    out_ref[0, 0, :] = jnp.maximum(correct - best_other, 0.0)


def _tc_tail_kernel(y_ref, x_ref, m_ref, c_ref):
    x = x_ref[...]
    yb = y_ref[0, 0, :]
    col = _SC_COLS + jax.lax.broadcasted_iota(jnp.int32, x.shape, 1)
    is_y = col == yb[:, None]
    valid = col < _COLS
    neg = jnp.float32(-jnp.inf)
    m_ref[0, 0, :] = jnp.max(jnp.where(valid & (~is_y), x, neg), axis=-1)
    c_ref[0, 0, :] = jnp.max(jnp.where(valid & is_y, x, neg), axis=-1)


@jax.jit
def kernel(logits, y):
    y32 = y.astype(jnp.int32)
    m_sc, c_sc = pl.kernel(
        _sc_body,
        out_type=(
            jax.ShapeDtypeStruct((_SC_ROWS,), jnp.float32),
            jax.ShapeDtypeStruct((_SC_ROWS,), jnp.float32),
        ),
        mesh=_MESH,
        scratch_types=[
            pltpu.VMEM((_RPW,), jnp.int32),
            pltpu.VMEM((8, _W), jnp.float32),
            pltpu.VMEM((8, _W), jnp.float32),
            pltpu.VMEM((8, _W), jnp.float32),
            pltpu.VMEM((8, _W), jnp.float32),
            pltpu.VMEM((_RPW,), jnp.float32),
            pltpu.VMEM((_RPW,), jnp.float32),
            pltpu.SemaphoreType.DMA((4,)),
        ],
    )(logits, y32)

    y3 = y32.reshape(_ROWS // _TC_BLOCK_ROWS, 1, _TC_BLOCK_ROWS)
    m_tc, c_tc = pl.pallas_call(
        _tc_tail_kernel,
        grid=(_TC_GRID,),
        in_specs=[
            pl.BlockSpec((1, 1, _TC_BLOCK_ROWS), lambda i: (i, 0, 0)),
            pl.BlockSpec((_TC_BLOCK_ROWS, _TC_COL_BLOCK), lambda i: (i, _TC_COL_INDEX)),
        ],
        out_specs=[
            pl.BlockSpec((1, 1, _TC_BLOCK_ROWS), lambda i: (i, 0, 0)),
            pl.BlockSpec((1, 1, _TC_BLOCK_ROWS), lambda i: (i, 0, 0)),
        ],
        out_shape=[
            jax.ShapeDtypeStruct((_TC_GRID, 1, _TC_BLOCK_ROWS), jnp.float32),
            jax.ShapeDtypeStruct((_TC_GRID, 1, _TC_BLOCK_ROWS), jnp.float32),
        ],
    )(y3, logits)
    m_tc = m_tc.reshape(_SC_ROWS)
    c_tc = c_tc.reshape(_SC_ROWS)

    out_hi = pl.pallas_call(
        _tc_full_kernel,
        grid=(_TC_HI_GRID,),
        in_specs=[
            pl.BlockSpec((1, 1, _TC_BLOCK_ROWS), lambda i: (i + _TC_GRID, 0, 0)),
            pl.BlockSpec((_TC_BLOCK_ROWS, _COLS), lambda i: (i + _TC_GRID, 0)),
        ],
        out_specs=pl.BlockSpec((1, 1, _TC_BLOCK_ROWS), lambda i: (i, 0, 0)),
        out_shape=jax.ShapeDtypeStruct(
            (_TC_HI_GRID, 1, _TC_BLOCK_ROWS), jnp.float32
        ),
    )(y3, logits).reshape(_ROWS - _SC_ROWS)

    correct = jnp.maximum(c_sc, c_tc)
    best_other = jnp.maximum(m_sc, m_tc)
    out_lo = jnp.maximum(correct - best_other, jnp.float32(0.0))
    return jnp.concatenate([out_lo, out_hi])
